# baseline (speedup 1.0000x reference)
import functools

import jax
import jax.numpy as jnp
from jax.experimental import pallas as pl

NT_A = 2048
NT_B = 2048
_BIG = 3.0e38


def _kernel_a(xyz1_ref, xyz2t_ref, feat1_ref, feat2_ref, w1a_ref, w1b_ref,
              b1_ref, x1_ref, stats_ref, *, S):
    nt = xyz1_ref.shape[1]
    a0 = xyz1_ref[0, :, 0:1]
    a1 = xyz1_ref[0, :, 1:2]
    a2 = xyz1_ref[0, :, 2:3]
    b0 = xyz2t_ref[0, 0:1, :]
    b1 = xyz2t_ref[0, 1:2, :]
    b2 = xyz2t_ref[0, 2:3, :]
    n1 = a0 * a0 + a1 * a1 + a2 * a2
    sq2 = b0 * b0 + b1 * b1 + b2 * b2
    cross = jnp.dot(xyz1_ref[0], xyz2t_ref[0],
                    preferred_element_type=jnp.float32)
    d2 = (n1 + sq2) - (cross + cross)

    m0 = jnp.min(d2, axis=1, keepdims=True)
    eq0 = d2 == m0
    dm1 = jnp.where(eq0, _BIG, d2)
    m1 = jnp.min(dm1, axis=1, keepdims=True)
    eq1 = dm1 == m1
    dm2 = jnp.where(eq1, _BIG, dm1)
    m2 = jnp.min(dm2, axis=1, keepdims=True)
    eq2 = dm2 == m2

    r0 = 1.0 / (jnp.sqrt(jnp.maximum(m0, 0.0)) + 1e-8)
    r1 = 1.0 / (jnp.sqrt(jnp.maximum(m1, 0.0)) + 1e-8)
    r2 = 1.0 / (jnp.sqrt(jnp.maximum(m2, 0.0)) + 1e-8)
    norm = r0 + r1 + r2
    zero = jnp.float32(0.0)
    wmat = jnp.where(eq2, r2 / norm, zero)
    wmat = jnp.where(eq1, r1 / norm, wmat)
    wmat = jnp.where(eq0, r0 / norm, wmat)

    interp = jax.lax.dot_general(
        wmat, feat2_ref[0], (((1,), (1,)), ((), ())),
        preferred_element_type=jnp.float32)
    x1 = (jnp.dot(interp, w1a_ref[...], preferred_element_type=jnp.float32)
          + jax.lax.dot_general(
              feat1_ref[0], w1b_ref[...], (((0,), (0,)), ((), ())),
              preferred_element_type=jnp.float32)
          + b1_ref[...])
    x1_ref[0] = x1

    @pl.when((pl.program_id(0) == 0) & (pl.program_id(1) == 0))
    def _init():
        stats_ref[...] = jnp.zeros_like(stats_ref)

    stats_ref[0:1, :] += jnp.sum(x1, axis=0, keepdims=True)
    stats_ref[1:2, :] += jnp.sum(x1 * x1, axis=0, keepdims=True)


def _bn_affine(stats_ref, g_ref, be_ref, inv_cnt):
    mean = stats_ref[0:1, :] * inv_cnt
    var = stats_ref[1:2, :] * inv_cnt - mean * mean
    a = g_ref[...] / jnp.sqrt(var + 1e-5)
    c = be_ref[...] - a * mean
    return a, c


def _kernel_b(x1_ref, st1_ref, g1_ref, be1_ref, w2t_ref, b2_ref, x2_ref,
              stats_ref, *, inv_cnt):
    a1, c1 = _bn_affine(st1_ref, g1_ref, be1_ref, inv_cnt)
    y1 = jnp.maximum(x1_ref[0] * a1 + c1, 0.0)
    x2 = (jnp.dot(y1, w2t_ref[...], preferred_element_type=jnp.float32)
          + b2_ref[...])
    x2_ref[0] = x2

    @pl.when((pl.program_id(0) == 0) & (pl.program_id(1) == 0))
    def _init():
        stats_ref[...] = jnp.zeros_like(stats_ref)

    stats_ref[0:1, :] += jnp.sum(x2, axis=0, keepdims=True)
    stats_ref[1:2, :] += jnp.sum(x2 * x2, axis=0, keepdims=True)


def _kernel_c(x2_ref, st2_ref, g2_ref, be2_ref, out_ref, *, inv_cnt):
    a2, c2 = _bn_affine(st2_ref, g2_ref, be2_ref, inv_cnt)
    y = jnp.maximum(x2_ref[0] * a2 + c2, 0.0)
    out_ref[0] = y.T


def kernel(xyz1, xyz2, feat1, feat2, W1, b1, g1, be1, W2, b2, g2, be2):
    B, N, _ = xyz1.shape
    S = xyz2.shape[1]
    C1 = feat1.shape[1]
    C2 = feat2.shape[1]
    M1 = W1.shape[0]
    M2 = W2.shape[0]
    f32 = jnp.float32

    xyz2t = xyz2.transpose(0, 2, 1)
    w1a = W1[:, :C2].T
    w1b = W1[:, C2:].T

    nta = NT_A
    ga = (B, N // nta)
    x1, stats1 = pl.pallas_call(
        functools.partial(_kernel_a, S=S),
        grid=ga,
        in_specs=[
            pl.BlockSpec((1, nta, 3), lambda b, i: (b, i, 0)),
            pl.BlockSpec((1, 3, S), lambda b, i: (b, 0, 0)),
            pl.BlockSpec((1, C1, nta), lambda b, i: (b, 0, i)),
            pl.BlockSpec((1, C2, S), lambda b, i: (b, 0, 0)),
            pl.BlockSpec((C2, M1), lambda b, i: (0, 0)),
            pl.BlockSpec((C1, M1), lambda b, i: (0, 0)),
            pl.BlockSpec((1, M1), lambda b, i: (0, 0)),
        ],
        out_specs=[
            pl.BlockSpec((1, nta, M1), lambda b, i: (b, i, 0)),
            pl.BlockSpec((8, M1), lambda b, i: (0, 0)),
        ],
        out_shape=[
            jax.ShapeDtypeStruct((B, N, M1), f32),
            jax.ShapeDtypeStruct((8, M1), f32),
        ],
    )(xyz1, xyz2t, feat1, feat2, w1a, w1b, b1.reshape(1, M1))

    inv_cnt = 1.0 / (B * N)
    ntb = NT_B
    gb = (B, N // ntb)
    x2, stats2 = pl.pallas_call(
        functools.partial(_kernel_b, inv_cnt=inv_cnt),
        grid=gb,
        in_specs=[
            pl.BlockSpec((1, ntb, M1), lambda b, i: (b, i, 0)),
            pl.BlockSpec((8, M1), lambda b, i: (0, 0)),
            pl.BlockSpec((1, M1), lambda b, i: (0, 0)),
            pl.BlockSpec((1, M1), lambda b, i: (0, 0)),
            pl.BlockSpec((M1, M2), lambda b, i: (0, 0)),
            pl.BlockSpec((1, M2), lambda b, i: (0, 0)),
        ],
        out_specs=[
            pl.BlockSpec((1, ntb, M2), lambda b, i: (b, i, 0)),
            pl.BlockSpec((8, M2), lambda b, i: (0, 0)),
        ],
        out_shape=[
            jax.ShapeDtypeStruct((B, N, M2), f32),
            jax.ShapeDtypeStruct((8, M2), f32),
        ],
    )(x1, stats1, g1.reshape(1, M1), be1.reshape(1, M1), W2.T,
      b2.reshape(1, M2))

    out = pl.pallas_call(
        functools.partial(_kernel_c, inv_cnt=inv_cnt),
        grid=gb,
        in_specs=[
            pl.BlockSpec((1, ntb, M2), lambda b, i: (b, i, 0)),
            pl.BlockSpec((8, M2), lambda b, i: (0, 0)),
            pl.BlockSpec((1, M2), lambda b, i: (0, 0)),
            pl.BlockSpec((1, M2), lambda b, i: (0, 0)),
        ],
        out_specs=pl.BlockSpec((1, M2, ntb), lambda b, i: (b, 0, i)),
        out_shape=jax.ShapeDtypeStruct((B, M2, N), f32),
    )(x2, stats2, g2.reshape(1, M2), be2.reshape(1, M2))
    return out


# speedup vs baseline: 1.0599x; 1.0599x over previous
import functools

import jax
import jax.numpy as jnp
from jax.experimental import pallas as pl

NT_A = 2048
NT_B = 8192
_BIG = 3.0e38


def _kernel_a(xyz1_ref, xyz2t_ref, feat1_ref, feat2_ref, w1a_ref, w1b_ref,
              b1_ref, x1_ref, stats_ref):
    a0 = xyz1_ref[0, :, 0:1]
    a1 = xyz1_ref[0, :, 1:2]
    a2 = xyz1_ref[0, :, 2:3]
    b0 = xyz2t_ref[0, 0:1, :]
    b1 = xyz2t_ref[0, 1:2, :]
    b2 = xyz2t_ref[0, 2:3, :]
    d0 = a0 - b0
    d1 = a1 - b1
    d2c = a2 - b2
    d2 = d0 * d0 + d1 * d1 + d2c * d2c

    m0 = jnp.min(d2, axis=1, keepdims=True)
    eq0 = d2 == m0
    dm1 = jnp.where(eq0, _BIG, d2)
    m1 = jnp.min(dm1, axis=1, keepdims=True)
    eq1 = dm1 == m1
    dm2 = jnp.where(eq1, _BIG, dm1)
    m2 = jnp.min(dm2, axis=1, keepdims=True)
    eq2 = dm2 == m2

    r0 = 1.0 / (jnp.sqrt(m0) + 1e-8)
    r1 = 1.0 / (jnp.sqrt(m1) + 1e-8)
    r2 = 1.0 / (jnp.sqrt(m2) + 1e-8)
    norm = r0 + r1 + r2
    zero = jnp.float32(0.0)
    wmat = jnp.where(eq2, r2 / norm, zero)
    wmat = jnp.where(eq1, r1 / norm, wmat)
    wmat = jnp.where(eq0, r0 / norm, wmat)

    interp = jax.lax.dot_general(
        wmat, feat2_ref[0], (((1,), (1,)), ((), ())),
        preferred_element_type=jnp.float32)
    x1 = (jnp.dot(interp, w1a_ref[...], preferred_element_type=jnp.float32)
          + jax.lax.dot_general(
              feat1_ref[0], w1b_ref[...], (((0,), (0,)), ((), ())),
              preferred_element_type=jnp.float32)
          + b1_ref[...])
    x1_ref[0] = x1

    @pl.when((pl.program_id(0) == 0) & (pl.program_id(1) == 0))
    def _init():
        stats_ref[...] = jnp.zeros_like(stats_ref)

    stats_ref[0:1, :] += jnp.sum(x1, axis=0, keepdims=True)
    stats_ref[1:2, :] += jnp.sum(x1 * x1, axis=0, keepdims=True)


def _bn_affine(stats_ref, g_ref, be_ref, inv_cnt):
    mean = stats_ref[0:1, :] * inv_cnt
    var = stats_ref[1:2, :] * inv_cnt - mean * mean
    a = g_ref[...] / jnp.sqrt(var + 1e-5)
    c = be_ref[...] - a * mean
    return a, c


def _kernel_b(x1_ref, st1_ref, g1_ref, be1_ref, w2t_ref, b2_ref, x2_ref,
              stats_ref, *, inv_cnt):
    a1, c1 = _bn_affine(st1_ref, g1_ref, be1_ref, inv_cnt)
    y1 = jnp.maximum(x1_ref[0] * a1 + c1, 0.0)
    x2 = (jnp.dot(y1, w2t_ref[...], preferred_element_type=jnp.float32)
          + b2_ref[...])
    x2_ref[0] = x2

    @pl.when((pl.program_id(0) == 0) & (pl.program_id(1) == 0))
    def _init():
        stats_ref[...] = jnp.zeros_like(stats_ref)

    stats_ref[0:1, :] += jnp.sum(x2, axis=0, keepdims=True)
    stats_ref[1:2, :] += jnp.sum(x2 * x2, axis=0, keepdims=True)


def _kernel_c(x2_ref, st2_ref, g2_ref, be2_ref, out_ref, *, inv_cnt):
    a2, c2 = _bn_affine(st2_ref, g2_ref, be2_ref, inv_cnt)
    y = jnp.maximum(x2_ref[0] * a2 + c2, 0.0)
    out_ref[0] = y.T


def kernel(xyz1, xyz2, feat1, feat2, W1, b1, g1, be1, W2, b2, g2, be2):
    B, N, _ = xyz1.shape
    S = xyz2.shape[1]
    C1 = feat1.shape[1]
    C2 = feat2.shape[1]
    M1 = W1.shape[0]
    M2 = W2.shape[0]
    f32 = jnp.float32

    xyz2t = xyz2.transpose(0, 2, 1)
    w1a = W1[:, :C2].T
    w1b = W1[:, C2:].T

    nta = min(NT_A, N)
    ga = (B, N // nta)
    x1, stats1 = pl.pallas_call(
        _kernel_a,
        grid=ga,
        in_specs=[
            pl.BlockSpec((1, nta, 3), lambda b, i: (b, i, 0)),
            pl.BlockSpec((1, 3, S), lambda b, i: (b, 0, 0)),
            pl.BlockSpec((1, C1, nta), lambda b, i: (b, 0, i)),
            pl.BlockSpec((1, C2, S), lambda b, i: (b, 0, 0)),
            pl.BlockSpec((C2, M1), lambda b, i: (0, 0)),
            pl.BlockSpec((C1, M1), lambda b, i: (0, 0)),
            pl.BlockSpec((1, M1), lambda b, i: (0, 0)),
        ],
        out_specs=[
            pl.BlockSpec((1, nta, M1), lambda b, i: (b, i, 0)),
            pl.BlockSpec((8, M1), lambda b, i: (0, 0)),
        ],
        out_shape=[
            jax.ShapeDtypeStruct((B, N, M1), f32),
            jax.ShapeDtypeStruct((8, M1), f32),
        ],
    )(xyz1, xyz2t, feat1, feat2, w1a, w1b, b1.reshape(1, M1))

    inv_cnt = 1.0 / (B * N)
    ntb = min(NT_B, N)
    gb = (B, N // ntb)
    x2, stats2 = pl.pallas_call(
        functools.partial(_kernel_b, inv_cnt=inv_cnt),
        grid=gb,
        in_specs=[
            pl.BlockSpec((1, ntb, M1), lambda b, i: (b, i, 0)),
            pl.BlockSpec((8, M1), lambda b, i: (0, 0)),
            pl.BlockSpec((1, M1), lambda b, i: (0, 0)),
            pl.BlockSpec((1, M1), lambda b, i: (0, 0)),
            pl.BlockSpec((M1, M2), lambda b, i: (0, 0)),
            pl.BlockSpec((1, M2), lambda b, i: (0, 0)),
        ],
        out_specs=[
            pl.BlockSpec((1, ntb, M2), lambda b, i: (b, i, 0)),
            pl.BlockSpec((8, M2), lambda b, i: (0, 0)),
        ],
        out_shape=[
            jax.ShapeDtypeStruct((B, N, M2), f32),
            jax.ShapeDtypeStruct((8, M2), f32),
        ],
    )(x1, stats1, g1.reshape(1, M1), be1.reshape(1, M1), W2.T,
      b2.reshape(1, M2))

    out = pl.pallas_call(
        functools.partial(_kernel_c, inv_cnt=inv_cnt),
        grid=gb,
        in_specs=[
            pl.BlockSpec((1, ntb, M2), lambda b, i: (b, i, 0)),
            pl.BlockSpec((8, M2), lambda b, i: (0, 0)),
            pl.BlockSpec((1, M2), lambda b, i: (0, 0)),
            pl.BlockSpec((1, M2), lambda b, i: (0, 0)),
        ],
        out_specs=pl.BlockSpec((1, M2, ntb), lambda b, i: (b, 0, i)),
        out_shape=jax.ShapeDtypeStruct((B, M2, N), f32),
    )(x2, stats2, g2.reshape(1, M2), be2.reshape(1, M2))
    return out
